# baseline (speedup 1.0000x reference)
import functools

import jax
import jax.numpy as jnp
from jax import lax
from jax.experimental import pallas as pl
from jax.experimental.pallas import tpu as pltpu
from jax.experimental.pallas import tpu_sc as plsc

NUM_ITEMS = 1000000
LATENT_DIM = 16
BATCH = 16384

NC = 2
NS = 16
NW = NC * NS
B_PER_W = BATCH // NW
CHUNK = 128
NCHUNK = B_PER_W // CHUNK
BLOCKS = B_PER_W // LATENT_DIM

SUB = 2048
NSUB = 8
N_PER_W = SUB * NSUB
SCN = N_PER_W * NW
TCN = NUM_ITEMS - SCN

TC_BLK = 131072
TC_GRID = (TCN + TC_BLK - 1) // TC_BLK
TC_OFF_BLKS = SCN // TC_BLK

_mesh = plsc.VectorSubcoreMesh(
    core_axis_name="c", subcore_axis_name="s", num_cores=NC, num_subcores=NS
)


@functools.partial(
    pl.kernel,
    out_type=jax.ShapeDtypeStruct((SCN,), jnp.float32),
    mesh=_mesh,
    compiler_params=pltpu.CompilerParams(
        needs_layout_passes=False, use_tc_tiling_on_sc=True
    ),
    scratch_types=[
        pltpu.VMEM((8, SUB), jnp.float32),
        pltpu.VMEM((8, SUB), jnp.float32),
        pltpu.VMEM((SUB,), jnp.float32),
        pltpu.VMEM((2 * LATENT_DIM,), jnp.float32),
    ],
)
def _sc_compute(tabT_hbm, params_hbm, y_hbm, t0_v, t1_v, y_v, params_v):
    wid = lax.axis_index("s") * NC + lax.axis_index("c")
    base = wid * N_PER_W
    pltpu.sync_copy(params_hbm, params_v)
    w_cols = [
        plsc.load_gather(params_v, [jnp.full((16,), LATENT_DIM + d, jnp.int32)])
        for d in range(LATENT_DIM)
    ]

    for k in range(NSUB):
        off = base + k * SUB
        pltpu.sync_copy(tabT_hbm.at[pl.ds(0, 8), pl.ds(off, SUB)], t0_v)
        pltpu.sync_copy(tabT_hbm.at[pl.ds(8, 8), pl.ds(off, SUB)], t1_v)

        def body(j, carry):
            i0 = j * 16
            acc = t0_v[0, pl.ds(i0, 16)] * w_cols[0]
            for s in range(1, 8):
                acc = acc + t0_v[s, pl.ds(i0, 16)] * w_cols[s]
            for s in range(8):
                acc = acc + t1_v[s, pl.ds(i0, 16)] * w_cols[8 + s]
            y_v[pl.ds(i0, 16)] = acc
            return carry

        lax.fori_loop(0, SUB // 16, body, 0)
        pltpu.sync_copy(y_v, y_hbm.at[pl.ds(off, SUB)])


def _tc_body(t_ref, w_ref, y_ref):
    t = t_ref[...]
    w = w_ref[...][:, 0:1]
    y_ref[...] = jnp.sum(t * w, axis=0)


_tc_matvec = pl.pallas_call(
    _tc_body,
    out_shape=jax.ShapeDtypeStruct((TCN,), jnp.float32),
    grid=(TC_GRID,),
    in_specs=[
        pl.BlockSpec((LATENT_DIM, TC_BLK), lambda g: (0, g + TC_OFF_BLKS)),
        pl.BlockSpec((LATENT_DIM, 128), lambda g: (0, 0)),
    ],
    out_specs=pl.BlockSpec((TC_BLK,), lambda g: (g,)),
)


@functools.partial(
    pl.kernel,
    out_type=jax.ShapeDtypeStruct((BATCH,), jnp.float32),
    mesh=_mesh,
    compiler_params=pltpu.CompilerParams(
        needs_layout_passes=False, use_tc_tiling_on_sc=False
    ),
    scratch_types=[
        pltpu.VMEM((B_PER_W,), jnp.int32),
        pltpu.VMEM((B_PER_W,), jnp.int32),
        pltpu.VMEM((B_PER_W,), jnp.int32),
        pltpu.VMEM((B_PER_W,), jnp.float32),
        pltpu.VMEM((B_PER_W,), jnp.float32),
        pltpu.VMEM((2 * LATENT_DIM,), jnp.float32),
        pltpu.VMEM((B_PER_W,), jnp.float32),
        pltpu.SemaphoreType.DMA,
    ],
)
def _sc_gather(idx_hbm, ysc_hbm, ytc_hbm, params_hbm, out_hbm,
               idx_v, i1_v, i2_v, g1_v, g2_v, params_v, out_v, sem):
    wid = lax.axis_index("s") * NC + lax.axis_index("c")
    base = wid * B_PER_W

    pltpu.sync_copy(idx_hbm.at[pl.ds(base, B_PER_W)], idx_v)
    pltpu.sync_copy(params_hbm, params_v)

    def clamp_body(j, carry):
        i0 = j * 16
        v = idx_v[pl.ds(i0, 16)]
        i1_v[pl.ds(i0, 16)] = jnp.minimum(v, SCN - 1)
        i2_v[pl.ds(i0, 16)] = jnp.clip(v - SCN, 0, TCN - 1)
        return carry

    lax.fori_loop(0, BLOCKS, clamp_body, 0)

    copies = []
    for c in range(NCHUNK):
        copies.append(pltpu.make_async_copy(
            ysc_hbm.at[i1_v.at[pl.ds(c * CHUNK, CHUNK)]],
            g1_v.at[pl.ds(c * CHUNK, CHUNK)],
            sem,
        ))
        copies.append(pltpu.make_async_copy(
            ytc_hbm.at[i2_v.at[pl.ds(c * CHUNK, CHUNK)]],
            g2_v.at[pl.ds(c * CHUNK, CHUNK)],
            sem,
        ))
    for cp in copies:
        cp.start()
    for cp in copies:
        cp.wait()

    b_vec = params_v[pl.ds(0, 16)]

    def body(j, carry):
        i0 = j * 16
        v = idx_v[pl.ds(i0, 16)]
        g = jnp.where(v < SCN, g1_v[pl.ds(i0, 16)], g2_v[pl.ds(i0, 16)])
        acc = g + b_vec
        out_v[pl.ds(i0, 16)] = 1.0 / (1.0 + jnp.exp(-acc))
        return carry

    lax.fori_loop(0, BLOCKS, body, 0)

    pltpu.sync_copy(out_v, out_hbm.at[pl.ds(base, B_PER_W)])


def kernel(item_indices, embedding_item, affine_W, affine_b):
    idx = item_indices.astype(jnp.int32)
    tabT = embedding_item.T
    params = jnp.concatenate(
        [jnp.broadcast_to(affine_b.reshape(1), (LATENT_DIM,)),
         affine_W.reshape(LATENT_DIM)]
    )
    w128 = jnp.broadcast_to(affine_W, (LATENT_DIM, 128))
    y_sc = _sc_compute(tabT, params)
    y_tc = _tc_matvec(tabT, w128)
    out = _sc_gather(idx, y_sc, y_tc, params)
    return out.reshape(BATCH, 1)


# speedup vs baseline: 2.2313x; 2.2313x over previous
import functools

import jax
import jax.numpy as jnp
from jax import lax
from jax.experimental import pallas as pl
from jax.experimental.pallas import tpu as pltpu
from jax.experimental.pallas import tpu_sc as plsc

NUM_ITEMS = 1000000
LATENT_DIM = 16
BATCH = 16384

NC = 2
NS = 16
NW = NC * NS
B_PER_W = BATCH // NW
CHUNK = 128
NCHUNK = B_PER_W // CHUNK
BLOCKS = B_PER_W // LATENT_DIM

SUB = 2048
NSUB = 6
N_PER_W = SUB * NSUB
SCN = N_PER_W * NW
TCN = NUM_ITEMS - SCN

TC_BLK = 131072
TC_GRID = (TCN + TC_BLK - 1) // TC_BLK
TC_OFF_BLKS = SCN // TC_BLK

_mesh = plsc.VectorSubcoreMesh(
    core_axis_name="c", subcore_axis_name="s", num_cores=NC, num_subcores=NS
)


@functools.partial(
    pl.kernel,
    out_type=jax.ShapeDtypeStruct((SCN,), jnp.float32),
    mesh=_mesh,
    compiler_params=pltpu.CompilerParams(
        needs_layout_passes=False, use_tc_tiling_on_sc=True
    ),
    scratch_types=[
        pltpu.VMEM((8, SUB), jnp.float32),
        pltpu.VMEM((8, SUB), jnp.float32),
        pltpu.VMEM((8, SUB), jnp.float32),
        pltpu.VMEM((8, SUB), jnp.float32),
        pltpu.VMEM((SUB,), jnp.float32),
        pltpu.VMEM((SUB,), jnp.float32),
        pltpu.VMEM((2 * LATENT_DIM,), jnp.float32),
        pltpu.SemaphoreType.DMA,
        pltpu.SemaphoreType.DMA,
        pltpu.SemaphoreType.DMA,
    ],
)
def _sc_compute(tabT_hbm, params_hbm, y_hbm,
                t0a_v, t0b_v, t1a_v, t1b_v, ya_v, yb_v, params_v,
                sem_a, sem_b, sem_y):
    wid = lax.axis_index("s") * NC + lax.axis_index("c")
    base = wid * N_PER_W
    pltpu.sync_copy(params_hbm, params_v)
    w_cols = [
        plsc.load_gather(params_v, [jnp.full((16,), LATENT_DIM + d, jnp.int32)])
        for d in range(LATENT_DIM)
    ]

    sems = [sem_a, sem_b]
    t0_bufs = [t0a_v, t0b_v]
    t1_bufs = [t1a_v, t1b_v]
    y_bufs = [ya_v, yb_v]

    def loads(k):
        p = k % 2
        off = base + k * SUB
        return [
            pltpu.make_async_copy(
                tabT_hbm.at[pl.ds(0, 8), pl.ds(off, SUB)], t0_bufs[p], sems[p]),
            pltpu.make_async_copy(
                tabT_hbm.at[pl.ds(8, 8), pl.ds(off, SUB)], t1_bufs[p], sems[p]),
        ]

    pending = loads(0)
    for cp in pending:
        cp.start()

    y_writes = [None, None]
    for k in range(NSUB):
        p = k % 2
        if k + 1 < NSUB:
            nxt = loads(k + 1)
            for cp in nxt:
                cp.start()
        for cp in pending:
            cp.wait()
        pending = nxt if k + 1 < NSUB else []

        t0 = t0_bufs[p]
        t1 = t1_bufs[p]
        yb = y_bufs[p]

        if y_writes[p] is not None:
            y_writes[p].wait()

        @plsc.parallel_loop(0, SUB // 16, unroll=4)
        def _(j):
            i0 = j * 16
            acc = t0[0, pl.ds(i0, 16)] * w_cols[0]
            for s in range(1, 8):
                acc = acc + t0[s, pl.ds(i0, 16)] * w_cols[s]
            for s in range(8):
                acc = acc + t1[s, pl.ds(i0, 16)] * w_cols[8 + s]
            yb[pl.ds(i0, 16)] = acc

        wr = pltpu.make_async_copy(
            yb, y_hbm.at[pl.ds(base + k * SUB, SUB)], sem_y)
        wr.start()
        y_writes[p] = wr

    for wr in y_writes:
        if wr is not None:
            wr.wait()


def _tc_body(t_ref, w_ref, y_ref):
    t = t_ref[...]
    w = w_ref[...][:, 0:1]
    y_ref[...] = jnp.sum(t * w, axis=0)


_tc_matvec = pl.pallas_call(
    _tc_body,
    out_shape=jax.ShapeDtypeStruct((TCN,), jnp.float32),
    grid=(TC_GRID,),
    in_specs=[
        pl.BlockSpec((LATENT_DIM, TC_BLK), lambda g: (0, g + TC_OFF_BLKS)),
        pl.BlockSpec((LATENT_DIM, 128), lambda g: (0, 0)),
    ],
    out_specs=pl.BlockSpec((TC_BLK,), lambda g: (g,)),
)


@functools.partial(
    pl.kernel,
    out_type=jax.ShapeDtypeStruct((BATCH,), jnp.float32),
    mesh=_mesh,
    compiler_params=pltpu.CompilerParams(
        needs_layout_passes=False, use_tc_tiling_on_sc=False
    ),
    scratch_types=[
        pltpu.VMEM((NCHUNK, CHUNK), jnp.int32),
        pltpu.VMEM((B_PER_W,), jnp.float32),
        pltpu.VMEM((LATENT_DIM,), jnp.float32),
        pltpu.VMEM((B_PER_W,), jnp.float32),
        pltpu.SemaphoreType.DMA,
    ],
)
def _sc_gather(idx_hbm, y_hbm, b_hbm, out_hbm,
               idx_v, g_v, b_v, out_v, sem):
    wid = lax.axis_index("s") * NC + lax.axis_index("c")
    base = wid * B_PER_W

    pltpu.sync_copy(idx_hbm.at[pl.ds(wid * NCHUNK, NCHUNK)], idx_v)
    pltpu.sync_copy(b_hbm, b_v)

    copies = [
        pltpu.make_async_copy(
            y_hbm.at[idx_v.at[c]],
            g_v.at[pl.ds(c * CHUNK, CHUNK)],
            sem,
        )
        for c in range(NCHUNK)
    ]
    for cp in copies:
        cp.start()
    for cp in copies:
        cp.wait()

    b_vec = b_v[...]

    def body(j, carry):
        i0 = j * 16
        acc = g_v[pl.ds(i0, 16)] + b_vec
        out_v[pl.ds(i0, 16)] = 1.0 / (1.0 + jnp.exp(-acc))
        return carry

    lax.fori_loop(0, BLOCKS, body, 0)

    pltpu.sync_copy(out_v, out_hbm.at[pl.ds(base, B_PER_W)])


def kernel(item_indices, embedding_item, affine_W, affine_b):
    idx2 = item_indices.astype(jnp.int32).reshape(NW * NCHUNK, CHUNK)
    tabT = embedding_item.T
    params = jnp.concatenate(
        [jnp.broadcast_to(affine_b.reshape(1), (LATENT_DIM,)),
         affine_W.reshape(LATENT_DIM)]
    )
    w128 = jnp.broadcast_to(affine_W, (LATENT_DIM, 128))
    y_sc = _sc_compute(tabT, params)
    y_tc = _tc_matvec(tabT, w128)
    y = jnp.concatenate([y_sc, y_tc])
    b16 = jnp.broadcast_to(affine_b.reshape(1), (LATENT_DIM,))
    out = _sc_gather(idx2, y, b16)
    return out.reshape(BATCH, 1)


# speedup vs baseline: 2.4984x; 1.1197x over previous
import functools

import jax
import jax.numpy as jnp
from jax import lax
from jax.experimental import pallas as pl
from jax.experimental.pallas import tpu as pltpu
from jax.experimental.pallas import tpu_sc as plsc

NUM_ITEMS = 1000000
LATENT_DIM = 16
BATCH = 16384

NC = 2
NS = 16
NW = NC * NS
B_PER_W = BATCH // NW
CHUNK = 128
NCHUNK = B_PER_W // CHUNK
BLOCKS = B_PER_W // LATENT_DIM

TC_BLK = 131072
TC_GRID = (NUM_ITEMS + TC_BLK - 1) // TC_BLK


def _tc_body(t_ref, w_ref, y_ref):
    t = t_ref[...]
    w = w_ref[...][:, 0:1]
    y_ref[...] = jnp.sum(t * w, axis=0)


_tc_matvec = pl.pallas_call(
    _tc_body,
    out_shape=jax.ShapeDtypeStruct((NUM_ITEMS,), jnp.float32),
    grid=(TC_GRID,),
    in_specs=[
        pl.BlockSpec((LATENT_DIM, TC_BLK), lambda g: (0, g)),
        pl.BlockSpec((LATENT_DIM, 128), lambda g: (0, 0)),
    ],
    out_specs=pl.BlockSpec((TC_BLK,), lambda g: (g,)),
)

_mesh = plsc.VectorSubcoreMesh(
    core_axis_name="c", subcore_axis_name="s", num_cores=NC, num_subcores=NS
)


@functools.partial(
    pl.kernel,
    out_type=jax.ShapeDtypeStruct((BATCH,), jnp.float32),
    mesh=_mesh,
    compiler_params=pltpu.CompilerParams(
        needs_layout_passes=False, use_tc_tiling_on_sc=False
    ),
    scratch_types=[
        pltpu.VMEM((NCHUNK, CHUNK), jnp.int32),
        pltpu.VMEM((B_PER_W,), jnp.float32),
        pltpu.VMEM((LATENT_DIM,), jnp.float32),
        pltpu.VMEM((B_PER_W,), jnp.float32),
        pltpu.SemaphoreType.DMA,
    ],
)
def _sc_gather(idx_hbm, y_hbm, b_hbm, out_hbm,
               idx_v, g_v, b_v, out_v, sem):
    wid = lax.axis_index("s") * NC + lax.axis_index("c")
    base = wid * B_PER_W

    pltpu.sync_copy(idx_hbm.at[pl.ds(wid * NCHUNK, NCHUNK)], idx_v)
    pltpu.sync_copy(b_hbm, b_v)

    copies = [
        pltpu.make_async_copy(
            y_hbm.at[idx_v.at[c]],
            g_v.at[pl.ds(c * CHUNK, CHUNK)],
            sem,
        )
        for c in range(NCHUNK)
    ]
    for cp in copies:
        cp.start()
    for cp in copies:
        cp.wait()

    b_vec = b_v[...]

    def body(j, carry):
        i0 = j * 16
        acc = g_v[pl.ds(i0, 16)] + b_vec
        out_v[pl.ds(i0, 16)] = 1.0 / (1.0 + jnp.exp(-acc))
        return carry

    lax.fori_loop(0, BLOCKS, body, 0)

    pltpu.sync_copy(out_v, out_hbm.at[pl.ds(base, B_PER_W)])


def kernel(item_indices, embedding_item, affine_W, affine_b):
    idx2 = item_indices.astype(jnp.int32).reshape(NW * NCHUNK, CHUNK)
    w128 = jnp.broadcast_to(affine_W, (LATENT_DIM, 128))
    y = _tc_matvec(embedding_item.T, w128)
    b16 = jnp.broadcast_to(affine_b.reshape(1), (LATENT_DIM,))
    out = _sc_gather(idx2, y, b16)
    return out.reshape(BATCH, 1)
